# Initial kernel scaffold; baseline (speedup 1.0000x reference)
import jax
import jax.numpy as jnp
from jax.experimental import pallas as pl


def kernel(output, target, heatmap):
    raise NotImplementedError("write your pallas kernel here")


# speedup vs baseline: 1.3605x; 1.3605x over previous
import jax
import jax.numpy as jnp
from jax.experimental import pallas as pl
from jax.experimental.pallas import tpu as pltpu


def _loss_body(o_ref, t_ref, h_ref, out_ref, acc_ref):
    i = pl.program_id(1)

    @pl.when(i == 0)
    def _init():
        acc_ref[...] = jnp.zeros_like(acc_ref)

    h = h_ref[...]
    m = jnp.max(h, axis=-1, keepdims=True)
    e = jnp.exp(h - m)
    denom = jnp.sum(e, axis=-1, keepdims=True)
    w = e * pl.reciprocal(denom, approx=False)

    o = o_ref[...]
    t = t_ref[...]
    s = jnp.sum(jnp.abs(o - t), axis=1)
    acc_ref[...] += jnp.sum(s * w)

    @pl.when(i == pl.num_programs(1) - 1)
    def _final():
        out_ref[...] = acc_ref[...].reshape(out_ref.shape)


def _pick_grid(n):
    for cores in (2, 1):
        if n % cores == 0:
            per_core = n // cores
            for bt in range(min(16, per_core), 0, -1):
                if per_core % bt == 0:
                    return cores, bt, per_core // bt
    return 1, 1, n


def kernel(output, target, heatmap):
    N, C, H, W = output.shape
    HW = H * W
    out_f = output.reshape(N, C, HW)
    tgt_f = target.reshape(N, C, HW)

    hm = jax.image.resize(heatmap.astype(jnp.float32), (N, 1, H, W),
                          method="bilinear", antialias=False)
    hm_f = hm.reshape(N, HW)

    cores, bt, inner = _pick_grid(N)

    partials = pl.pallas_call(
        _loss_body,
        out_shape=jax.ShapeDtypeStruct((cores, 1, 1), jnp.float32),
        grid=(cores, inner),
        in_specs=[
            pl.BlockSpec((bt, C, HW), lambda p, i: (p * inner + i, 0, 0)),
            pl.BlockSpec((bt, C, HW), lambda p, i: (p * inner + i, 0, 0)),
            pl.BlockSpec((bt, HW), lambda p, i: (p * inner + i, 0)),
        ],
        out_specs=pl.BlockSpec((1, 1, 1), lambda p, i: (p, 0, 0)),
        scratch_shapes=[pltpu.VMEM((1, 1), jnp.float32)],
        compiler_params=pltpu.CompilerParams(
            dimension_semantics=("parallel", "arbitrary")),
    )(out_f, tgt_f, hm_f)
    return jnp.sum(partials)


# speedup vs baseline: 1.3824x; 1.0161x over previous
import jax
import jax.numpy as jnp
from jax.experimental import pallas as pl
from jax.experimental.pallas import tpu as pltpu


def _loss_body(o_ref, t_ref, h_ref, out_ref, acc_ref):
    i = pl.program_id(1)

    @pl.when(i == 0)
    def _init():
        acc_ref[...] = jnp.zeros_like(acc_ref)

    h = h_ref[...]
    m = jnp.max(h, axis=-1, keepdims=True)
    e = jnp.exp(h - m)
    denom = jnp.sum(e, axis=-1, keepdims=True)
    w = e * pl.reciprocal(denom, approx=False)

    o = o_ref[...]
    t = t_ref[...]
    s = jnp.sum(jnp.abs(o - t), axis=1)
    acc_ref[...] += jnp.sum(s * w)

    @pl.when(i == pl.num_programs(1) - 1)
    def _final():
        out_ref[...] = acc_ref[...].reshape(out_ref.shape)


def _pick_grid(n):
    for cores in (2, 1):
        if n % cores == 0:
            per_core = n // cores
            for bt in range(min(32, per_core), 0, -1):
                if per_core % bt == 0:
                    return cores, bt, per_core // bt
    return 1, 1, n


def kernel(output, target, heatmap):
    N, C, H, W = output.shape
    HW = H * W
    out_f = output.reshape(N, C, HW)
    tgt_f = target.reshape(N, C, HW)

    hm = jax.image.resize(heatmap.astype(jnp.float32), (N, 1, H, W),
                          method="bilinear", antialias=False)
    hm_f = hm.reshape(N, HW)

    cores, bt, inner = _pick_grid(N)

    partials = pl.pallas_call(
        _loss_body,
        out_shape=jax.ShapeDtypeStruct((cores, 1, 1), jnp.float32),
        grid=(cores, inner),
        in_specs=[
            pl.BlockSpec((bt, C, HW), lambda p, i: (p * inner + i, 0, 0)),
            pl.BlockSpec((bt, C, HW), lambda p, i: (p * inner + i, 0, 0)),
            pl.BlockSpec((bt, HW), lambda p, i: (p * inner + i, 0)),
        ],
        out_specs=pl.BlockSpec((1, 1, 1), lambda p, i: (p, 0, 0)),
        scratch_shapes=[pltpu.VMEM((1, 1), jnp.float32)],
        compiler_params=pltpu.CompilerParams(
            dimension_semantics=("parallel", "arbitrary")),
    )(out_f, tgt_f, hm_f)
    return jnp.sum(partials)


# speedup vs baseline: 1.3935x; 1.0080x over previous
import jax
import jax.numpy as jnp
from jax.experimental import pallas as pl
from jax.experimental.pallas import tpu as pltpu


def _loss_body(o_ref, t_ref, h_ref, out_ref, acc_ref):
    i = pl.program_id(1)

    @pl.when(i == 0)
    def _init():
        acc_ref[...] = jnp.zeros_like(acc_ref)

    h = h_ref[...]
    m = jnp.max(h, axis=-1, keepdims=True)
    e = jnp.exp(h - m)
    denom = jnp.sum(e, axis=-1, keepdims=True)
    w = e * pl.reciprocal(denom, approx=False)

    o = o_ref[...]
    t = t_ref[...]
    s = jnp.sum(jnp.abs(o - t), axis=1)
    acc_ref[...] += jnp.sum(s * w)

    @pl.when(i == pl.num_programs(1) - 1)
    def _final():
        out_ref[...] = acc_ref[...].reshape(out_ref.shape)


def _pick_grid(n):
    for cores in (1,):
        if n % cores == 0:
            per_core = n // cores
            for bt in range(min(32, per_core), 0, -1):
                if per_core % bt == 0:
                    return cores, bt, per_core // bt
    return 1, 1, n


def kernel(output, target, heatmap):
    N, C, H, W = output.shape
    HW = H * W
    out_f = output.reshape(N, C, HW)
    tgt_f = target.reshape(N, C, HW)

    hm = jax.image.resize(heatmap.astype(jnp.float32), (N, 1, H, W),
                          method="bilinear", antialias=False)
    hm_f = hm.reshape(N, HW)

    cores, bt, inner = _pick_grid(N)

    partials = pl.pallas_call(
        _loss_body,
        out_shape=jax.ShapeDtypeStruct((cores, 1, 1), jnp.float32),
        grid=(cores, inner),
        in_specs=[
            pl.BlockSpec((bt, C, HW), lambda p, i: (p * inner + i, 0, 0)),
            pl.BlockSpec((bt, C, HW), lambda p, i: (p * inner + i, 0, 0)),
            pl.BlockSpec((bt, HW), lambda p, i: (p * inner + i, 0)),
        ],
        out_specs=pl.BlockSpec((1, 1, 1), lambda p, i: (p, 0, 0)),
        scratch_shapes=[pltpu.VMEM((1, 1), jnp.float32)],
        compiler_params=pltpu.CompilerParams(
            dimension_semantics=("parallel", "arbitrary")),
    )(out_f, tgt_f, hm_f)
    return jnp.sum(partials)


# speedup vs baseline: 1.4274x; 1.0243x over previous
import functools

import jax
import jax.numpy as jnp
import numpy as np
from jax.experimental import pallas as pl
from jax.experimental.pallas import tpu as pltpu


@functools.lru_cache(maxsize=None)
def _bilinear_matrix(dst, src):
    m = np.zeros((dst, src), np.float64)
    scale = src / dst
    for i in range(dst):
        c = (i + 0.5) * scale - 0.5
        lo = int(np.floor(c))
        f = c - lo
        m[i, min(max(lo, 0), src - 1)] += 1.0 - f
        m[i, min(max(lo + 1, 0), src - 1)] += f
    return jnp.asarray(m, jnp.float32)


def _loss_body(o_ref, t_ref, h_ref, out_ref, acc_ref):
    i = pl.program_id(1)

    @pl.when(i == 0)
    def _init():
        acc_ref[...] = jnp.zeros_like(acc_ref)

    h = h_ref[...]
    m = jnp.max(h, axis=-1, keepdims=True)
    e = jnp.exp(h - m)
    denom = jnp.sum(e, axis=-1, keepdims=True)
    w = e * pl.reciprocal(denom, approx=False)

    o = o_ref[...]
    t = t_ref[...]
    s = jnp.sum(jnp.abs(o - t), axis=1)
    acc_ref[...] += jnp.sum(s * w)

    @pl.when(i == pl.num_programs(1) - 1)
    def _final():
        out_ref[...] = acc_ref[...].reshape(out_ref.shape)


def _pick_grid(n):
    for cores in (1,):
        if n % cores == 0:
            per_core = n // cores
            for bt in range(min(32, per_core), 0, -1):
                if per_core % bt == 0:
                    return cores, bt, per_core // bt
    return 1, 1, n


def kernel(output, target, heatmap):
    N, C, H, W = output.shape
    HW = H * W
    out_f = output.reshape(N, C, HW)
    tgt_f = target.reshape(N, C, HW)

    hs, ws = heatmap.shape[2], heatmap.shape[3]
    mh = _bilinear_matrix(H, hs)
    mw = _bilinear_matrix(W, ws)
    hm32 = heatmap.reshape(N, hs, ws).astype(jnp.float32)
    t1 = jnp.einsum("hH,nHW->nhW", mh, hm32)
    hm_up = jnp.einsum("nhW,wW->nhw", t1, mw)
    hm_f = hm_up.reshape(N, HW)

    cores, bt, inner = _pick_grid(N)

    partials = pl.pallas_call(
        _loss_body,
        out_shape=jax.ShapeDtypeStruct((cores, 1, 1), jnp.float32),
        grid=(cores, inner),
        in_specs=[
            pl.BlockSpec((bt, C, HW), lambda p, i: (p * inner + i, 0, 0)),
            pl.BlockSpec((bt, C, HW), lambda p, i: (p * inner + i, 0, 0)),
            pl.BlockSpec((bt, HW), lambda p, i: (p * inner + i, 0)),
        ],
        out_specs=pl.BlockSpec((1, 1, 1), lambda p, i: (p, 0, 0)),
        scratch_shapes=[pltpu.VMEM((1, 1), jnp.float32)],
        compiler_params=pltpu.CompilerParams(
            dimension_semantics=("parallel", "arbitrary")),
    )(out_f, tgt_f, hm_f)
    return jnp.sum(partials)
